# baseline (speedup 1.0000x reference)
import functools

import jax
import jax.numpy as jnp
from jax import lax
from jax.experimental import pallas as pl
from jax.experimental.pallas import tpu as pltpu
from jax.experimental.pallas import tpu_sc as plsc

VOCAB_N = 1000000
D = 64
PAD = 0

NC = 2
NS = 16
NW = NC * NS

B, S = 4096, 200
IB = B // NW
LANES = 16
SCALE = 8.0
RING = 4


def _transpose_scale_chunk(idx_v, j, gbuf, obuf):
    iota = lax.iota(jnp.int32, LANES)
    kbs = [(c0 + iota) >> 3 for c0 in range(0, D, LANES)]
    kks = [(c0 + iota) & 7 for c0 in range(0, D, LANES)]

    @plsc.parallel_loop(0, IB, 4, unroll=8)
    def _(r0):
        for dr in range(4):
            r = r0 + dr
            col = jnp.full((LANES,), r, jnp.int32)
            for ci in range(D // LANES):
                vals = gbuf[r, pl.ds(ci * LANES, LANES)] * SCALE
                plsc.store_scatter(obuf, [kbs[ci], kks[ci], col], vals)

    @pl.loop(0, IB // LANES)
    def _(g):
        r0 = g * LANES
        pad = idx_v[j, pl.ds(r0, LANES)] == PAD

        @pl.when(jnp.any(pad))
        def _():
            zero = jnp.zeros((LANES,), jnp.float32)
            for kb in range(8):
                for kk in range(8):
                    cur = obuf[kb, kk, pl.ds(r0, LANES)]
                    obuf[kb, kk, pl.ds(r0, LANES)] = jnp.where(pad, zero, cur)


def _emb_kernel(idx_hbm, table_hbm, out_hbm, idx_v, gbufs, obufs, sem_g, sem_o):
    wid = lax.axis_index("c") * NS + lax.axis_index("s")

    pltpu.sync_copy(idx_hbm.at[:, pl.ds(wid * IB, IB)], idx_v)

    def gather(j, b):
        return pltpu.make_async_copy(
            table_hbm.at[idx_v.at[j]], gbufs.at[b], sem_g
        )

    def out_piece(j, b):
        return pltpu.make_async_copy(
            obufs.at[b, :, :, pl.ds(0, IB)], out_hbm.at[j, :, wid], sem_o
        )

    for b in range(RING):
        gather(b, b).start()

    @pl.loop(0, S // RING)
    def _(g):
        for b in range(RING):
            j = g * RING + b
            gather(j, b).wait()

            @pl.when(g > 0)
            def _():
                out_piece(j - RING, b).wait()

            _transpose_scale_chunk(idx_v, j, gbufs.at[b], obufs.at[b])

            out_piece(j, b).start()

            @pl.when(g < S // RING - 1)
            def _():
                gather(j + RING, b).start()

    for b in range(RING):
        out_piece(S - RING + b, b).wait()


@jax.jit
def _embed(idx_t, table):
    mesh = plsc.VectorSubcoreMesh(core_axis_name="c", subcore_axis_name="s")
    cp = pltpu.CompilerParams(
        needs_layout_passes=False, use_tc_tiling_on_sc=False
    )
    run = functools.partial(
        pl.kernel,
        mesh=mesh,
        compiler_params=cp,
        out_type=jax.ShapeDtypeStruct((S, 8, NW, 8, 2 * D), jnp.float32),
        scratch_types=[
            pltpu.VMEM((S, IB), jnp.int32),
            pltpu.VMEM((RING, IB, D), jnp.float32),
            pltpu.VMEM((RING, 8, 8, IB + 1), jnp.float32),
            pltpu.SemaphoreType.DMA,
            pltpu.SemaphoreType.DMA,
        ],
    )(_emb_kernel)
    return run(idx_t, table)


def kernel(input, lookup_table):
    idx_t = input.T
    out = _embed(idx_t, lookup_table)
    return out.transpose(2, 4, 0, 1, 3).reshape(B, S, D)


# speedup vs baseline: 1.1606x; 1.1606x over previous
import functools

import jax
import jax.numpy as jnp
from jax import lax
from jax.experimental import pallas as pl
from jax.experimental.pallas import tpu as pltpu
from jax.experimental.pallas import tpu_sc as plsc

VOCAB_N = 1000000
D = 64
PAD = 0

NC = 2
NS = 16
NW = NC * NS

B, S = 4096, 200
IB = B // NW
LANES = 16
SCALE = 8.0
RING = 4


def _transpose_scale_chunk(idx_v, j, gbuf, obuf):
    iota = lax.iota(jnp.int32, LANES)
    kbs = [(c0 + iota) >> 3 for c0 in range(0, D, LANES)]
    kks = [(c0 + iota) & 7 for c0 in range(0, D, LANES)]

    @plsc.parallel_loop(0, IB, 2, unroll=4)
    def _(r0):
        for dr in range(2):
            r = r0 + dr
            col = jnp.full((LANES,), r, jnp.int32)
            for ci in range(D // LANES):
                vals = gbuf[r, pl.ds(ci * LANES, LANES)] * SCALE
                plsc.store_scatter(obuf, [kbs[ci], kks[ci], col], vals)

    @pl.loop(0, IB // LANES)
    def _(g):
        r0 = g * LANES
        pad = idx_v[j, pl.ds(r0, LANES)] == PAD

        @pl.when(jnp.any(pad))
        def _():
            zero = jnp.zeros((LANES,), jnp.float32)
            for kb in range(8):
                for kk in range(8):
                    cur = obuf[kb, kk, pl.ds(r0, LANES)]
                    obuf[kb, kk, pl.ds(r0, LANES)] = jnp.where(pad, zero, cur)


def _emb_kernel(idx_hbm, table_hbm, out_hbm, idx_v, gbufs, obufs, sem_g, sem_o):
    wid = lax.axis_index("c") * NS + lax.axis_index("s")

    pltpu.sync_copy(idx_hbm.at[:, pl.ds(wid * IB, IB)], idx_v)

    def gather(j, b):
        return pltpu.make_async_copy(
            table_hbm.at[idx_v.at[j]], gbufs.at[b], sem_g
        )

    def out_piece(j, b):
        return pltpu.make_async_copy(
            obufs.at[b, :, :, pl.ds(0, IB)], out_hbm.at[j, :, wid], sem_o
        )

    for b in range(RING):
        gather(b, b).start()

    @pl.loop(0, S // RING)
    def _(g):
        for b in range(RING):
            j = g * RING + b
            gather(j, b).wait()

            @pl.when(g > 0)
            def _():
                out_piece(j - RING, b).wait()

            _transpose_scale_chunk(idx_v, j, gbufs.at[b], obufs.at[b])

            out_piece(j, b).start()

            @pl.when(g < S // RING - 1)
            def _():
                gather(j + RING, b).start()

    for b in range(RING):
        out_piece(S - RING + b, b).wait()


@jax.jit
def _embed(idx_t, table):
    mesh = plsc.VectorSubcoreMesh(core_axis_name="c", subcore_axis_name="s")
    cp = pltpu.CompilerParams(
        needs_layout_passes=False, use_tc_tiling_on_sc=False
    )
    run = functools.partial(
        pl.kernel,
        mesh=mesh,
        compiler_params=cp,
        out_type=jax.ShapeDtypeStruct((S, 8, NW, 8, 2 * D), jnp.float32),
        scratch_types=[
            pltpu.VMEM((S, IB), jnp.int32),
            pltpu.VMEM((RING, IB, D), jnp.float32),
            pltpu.VMEM((RING, 8, 8, IB + 1), jnp.float32),
            pltpu.SemaphoreType.DMA,
            pltpu.SemaphoreType.DMA,
        ],
    )(_emb_kernel)
    return run(idx_t, table)


def kernel(input, lookup_table):
    idx_t = input.T
    out = _embed(idx_t, lookup_table)
    return out.transpose(2, 4, 0, 1, 3).reshape(B, S, D)


# speedup vs baseline: 1.1631x; 1.0022x over previous
import functools

import jax
import jax.numpy as jnp
from jax import lax
from jax.experimental import pallas as pl
from jax.experimental.pallas import tpu as pltpu
from jax.experimental.pallas import tpu_sc as plsc

VOCAB_N = 1000000
D = 64
PAD = 0

NC = 2
NS = 16
NW = NC * NS

B, S = 4096, 200
IB = B // NW
LANES = 16
SCALE = 8.0
RING = 4


def _transpose_scale_chunk(idx_v, j, gbuf, obuf):
    iota = lax.iota(jnp.int32, LANES)
    kbs = [(c0 + iota) >> 3 for c0 in range(0, D, LANES)]
    kks = [(c0 + iota) & 7 for c0 in range(0, D, LANES)]

    @plsc.parallel_loop(0, IB, 1, unroll=8)
    def _(r0):
        for dr in range(1):
            r = r0 + dr
            col = jnp.full((LANES,), r, jnp.int32)
            for ci in range(D // LANES):
                vals = gbuf[r, pl.ds(ci * LANES, LANES)] * SCALE
                plsc.store_scatter(obuf, [kbs[ci], kks[ci], col], vals)

    @pl.loop(0, IB // LANES)
    def _(g):
        r0 = g * LANES
        pad = idx_v[j, pl.ds(r0, LANES)] == PAD

        @pl.when(jnp.any(pad))
        def _():
            zero = jnp.zeros((LANES,), jnp.float32)
            for kb in range(8):
                for kk in range(8):
                    cur = obuf[kb, kk, pl.ds(r0, LANES)]
                    obuf[kb, kk, pl.ds(r0, LANES)] = jnp.where(pad, zero, cur)


def _emb_kernel(idx_hbm, table_hbm, out_hbm, idx_v, gbufs, obufs, sem_g, sem_o):
    wid = lax.axis_index("c") * NS + lax.axis_index("s")

    pltpu.sync_copy(idx_hbm.at[:, pl.ds(wid * IB, IB)], idx_v)

    def gather(j, b):
        return pltpu.make_async_copy(
            table_hbm.at[idx_v.at[j]], gbufs.at[b], sem_g
        )

    def out_piece(j, b):
        return pltpu.make_async_copy(
            obufs.at[b, :, :, pl.ds(0, IB)], out_hbm.at[j, :, wid], sem_o
        )

    for b in range(RING):
        gather(b, b).start()

    @pl.loop(0, S // RING)
    def _(g):
        for b in range(RING):
            j = g * RING + b
            gather(j, b).wait()

            @pl.when(g > 0)
            def _():
                out_piece(j - RING, b).wait()

            _transpose_scale_chunk(idx_v, j, gbufs.at[b], obufs.at[b])

            out_piece(j, b).start()

            @pl.when(g < S // RING - 1)
            def _():
                gather(j + RING, b).start()

    for b in range(RING):
        out_piece(S - RING + b, b).wait()


@jax.jit
def _embed(idx_t, table):
    mesh = plsc.VectorSubcoreMesh(core_axis_name="c", subcore_axis_name="s")
    cp = pltpu.CompilerParams(
        needs_layout_passes=False, use_tc_tiling_on_sc=False
    )
    run = functools.partial(
        pl.kernel,
        mesh=mesh,
        compiler_params=cp,
        out_type=jax.ShapeDtypeStruct((S, 8, NW, 8, 2 * D), jnp.float32),
        scratch_types=[
            pltpu.VMEM((S, IB), jnp.int32),
            pltpu.VMEM((RING, IB, D), jnp.float32),
            pltpu.VMEM((RING, 8, 8, IB + 1), jnp.float32),
            pltpu.SemaphoreType.DMA,
            pltpu.SemaphoreType.DMA,
        ],
    )(_emb_kernel)
    return run(idx_t, table)


def kernel(input, lookup_table):
    idx_t = input.T
    out = _embed(idx_t, lookup_table)
    return out.transpose(2, 4, 0, 1, 3).reshape(B, S, D)
